# Initial kernel scaffold; baseline (speedup 1.0000x reference)
import jax
import jax.numpy as jnp
from jax.experimental import pallas as pl


def kernel(output, target, edge_index):
    raise NotImplementedError("write your pallas kernel here")


# speedup vs baseline: 3.8524x; 3.8524x over previous
import functools

import jax
import jax.numpy as jnp
from jax import lax
from jax.experimental import pallas as pl
from jax.experimental.pallas import tpu as pltpu
from jax.experimental.pallas import tpu_sc as plsc

N_NODES = 10000
NUM_CLASSES = 128
N_EDGES = 320000
MU = 0.1

NC = 2
NS = 16
NW = NC * NS

E_PER_W = N_EDGES // NW
E_CHK = 80
N_CHUNKS = E_PER_W // E_CHK

SUP_CHK = 16
N_SUP_CHUNKS = N_NODES // SUP_CHK
SUP_ITERS = (N_SUP_CHUNKS + NW - 1) // NW

VECS_PER_ROW = NUM_CLASSES // 16


def _make_kernel():
    mesh = plsc.VectorSubcoreMesh(core_axis_name="c", subcore_axis_name="s")

    @functools.partial(
        pl.kernel,
        mesh=mesh,
        out_type=[
            jax.ShapeDtypeStruct((NW, 16), jnp.float32),
            jax.ShapeDtypeStruct((NW, 16), jnp.float32),
        ],
        scratch_types=[
            pltpu.VMEM((E_CHK,), jnp.int32),
            pltpu.VMEM((E_CHK,), jnp.int32),
            pltpu.VMEM((E_CHK, NUM_CLASSES), jnp.float32),
            pltpu.VMEM((E_CHK, NUM_CLASSES), jnp.float32),
            pltpu.VMEM((SUP_CHK, NUM_CLASSES), jnp.float32),
            pltpu.VMEM((SUP_CHK,), jnp.int32),
            pltpu.VMEM((16,), jnp.float32),
            pltpu.VMEM((16,), jnp.float32),
            pltpu.SemaphoreType.DMA,
            pltpu.SemaphoreType.DMA,
        ],
    )
    def graph_loss_kernel(out_hbm, tgt_hbm, row_hbm, col_hbm,
                          sup_out, smooth_out,
                          ridx_v, cidx_v, rows_v, cols_v,
                          suprows_v, suptgt_v, supstage_v, smstage_v,
                          sem_a, sem_b):
        wid = lax.axis_index("s") * NC + lax.axis_index("c")
        lanes = lax.iota(jnp.int32, 16)
        zero = jnp.zeros((16,), jnp.float32)

        def sup_body(k, acc):
            chunk = wid + NW * k
            valid = chunk < N_SUP_CHUNKS
            base = jnp.minimum(chunk, N_SUP_CHUNKS - 1) * SUP_CHK
            pltpu.sync_copy(tgt_hbm.at[pl.ds(base, SUP_CHK)], suptgt_v)
            pltpu.async_copy(
                out_hbm.at[pl.ds(base, SUP_CHK)], suprows_v, sem_a
            ).wait()
            contrib = jnp.zeros((16,), jnp.float32)
            tvec = suptgt_v[...]
            for i in range(SUP_CHK):
                t = tvec[i]
                for j in range(VECS_PER_ROW):
                    blk = suprows_v[i, pl.ds(j * 16, 16)]
                    contrib = contrib + jnp.where(lanes + j * 16 == t, blk, 0.0)
            return acc + jnp.where(valid, contrib, 0.0)

        sup_acc = lax.fori_loop(0, SUP_ITERS, sup_body, zero)
        supstage_v[...] = sup_acc
        pltpu.sync_copy(supstage_v, sup_out.at[wid])

        def chunk_body(c, accs):
            base = wid * E_PER_W + c * E_CHK
            pltpu.sync_copy(row_hbm.at[pl.ds(base, E_CHK)], ridx_v)
            pltpu.sync_copy(col_hbm.at[pl.ds(base, E_CHK)], cidx_v)
            cp_a = pltpu.async_copy(out_hbm.at[ridx_v], rows_v, sem_a)
            cp_b = pltpu.async_copy(out_hbm.at[cidx_v], cols_v, sem_b)
            cp_a.wait()
            cp_b.wait()

            def edge_body(e, a):
                new = []
                for j in range(VECS_PER_ROW):
                    r = rows_v[e, pl.ds(j * 16, 16)]
                    cc = cols_v[e, pl.ds(j * 16, 16)]
                    d = r - cc
                    new.append(a[j] + d * d)
                return tuple(new)

            return lax.fori_loop(0, E_CHK, edge_body, accs)

        accs = lax.fori_loop(
            0, N_CHUNKS, chunk_body, tuple(zero for _ in range(VECS_PER_ROW))
        )
        total = accs[0]
        for j in range(1, VECS_PER_ROW):
            total = total + accs[j]
        smstage_v[...] = total
        pltpu.sync_copy(smstage_v, smooth_out.at[wid])

    return graph_loss_kernel


_graph_loss = _make_kernel()


@jax.jit
def kernel(output, target, edge_index):
    row = edge_index[0]
    col = edge_index[1]
    sup_p, smooth_p = _graph_loss(output, target, row, col)
    supervised = -jnp.sum(sup_p) / N_NODES
    smoothness = jnp.sum(smooth_p) / (N_EDGES * NUM_CLASSES)
    return supervised + MU * smoothness


# speedup vs baseline: 8.6980x; 2.2578x over previous
import functools

import jax
import jax.numpy as jnp
from jax import lax
from jax.experimental import pallas as pl
from jax.experimental.pallas import tpu as pltpu
from jax.experimental.pallas import tpu_sc as plsc

N_NODES = 10000
NUM_CLASSES = 128
N_EDGES = 320000
MU = 0.1

NC = 2
NS = 16
NW = NC * NS

E_PER_W = N_EDGES // NW
E_CHK = 80
N_CHUNKS = E_PER_W // E_CHK

SUP_CHK = 16
N_SUP_CHUNKS = N_NODES // SUP_CHK
SUP_ITERS = (N_SUP_CHUNKS + NW - 1) // NW
SUP_ROWS = SUP_ITERS * SUP_CHK

VECS_PER_ROW = NUM_CLASSES // 16


def _make_kernel():
    mesh = plsc.VectorSubcoreMesh(core_axis_name="c", subcore_axis_name="s")

    @functools.partial(
        pl.kernel,
        mesh=mesh,
        out_type=[
            jax.ShapeDtypeStruct((NW, 16), jnp.float32),
            jax.ShapeDtypeStruct((NW, 16), jnp.float32),
        ],
        scratch_types=[
            pltpu.VMEM((E_PER_W,), jnp.int32),
            pltpu.VMEM((E_PER_W,), jnp.int32),
            pltpu.VMEM((E_CHK, NUM_CLASSES), jnp.float32),
            pltpu.VMEM((E_CHK, NUM_CLASSES), jnp.float32),
            pltpu.VMEM((E_CHK, NUM_CLASSES), jnp.float32),
            pltpu.VMEM((E_CHK, NUM_CLASSES), jnp.float32),
            pltpu.VMEM((SUP_ROWS, NUM_CLASSES), jnp.float32),
            pltpu.VMEM((SUP_ROWS,), jnp.int32),
            pltpu.VMEM((16,), jnp.float32),
            pltpu.VMEM((16,), jnp.float32),
            pltpu.SemaphoreType.DMA,
            pltpu.SemaphoreType.DMA,
            pltpu.SemaphoreType.DMA,
        ],
    )
    def graph_loss_kernel(out_hbm, tgt_hbm, row_hbm, col_hbm,
                          sup_out, smooth_out,
                          ridx_v, cidx_v, rows0_v, rows1_v, cols0_v, cols1_v,
                          suprows_v, suptgt_v, supstage_v, smstage_v,
                          sem_s, sem_g0, sem_g1):
        wid = lax.axis_index("s") * NC + lax.axis_index("c")
        lanes = lax.iota(jnp.int32, 16)
        zero = jnp.zeros((16,), jnp.float32)
        rows_b = (rows0_v, rows1_v)
        cols_b = (cols0_v, cols1_v)
        sem_g = (sem_g0, sem_g1)

        for k in range(SUP_ITERS):
            chunk = wid + NW * k
            base = jnp.minimum(chunk, N_SUP_CHUNKS - 1) * SUP_CHK
            pltpu.async_copy(
                tgt_hbm.at[pl.ds(base, SUP_CHK)],
                suptgt_v.at[pl.ds(k * SUP_CHK, SUP_CHK)], sem_s)
            pltpu.async_copy(
                out_hbm.at[pl.ds(base, SUP_CHK)],
                suprows_v.at[pl.ds(k * SUP_CHK, SUP_CHK)], sem_s)

        ebase = wid * E_PER_W
        pltpu.sync_copy(row_hbm.at[pl.ds(ebase, E_PER_W)], ridx_v)
        pltpu.sync_copy(col_hbm.at[pl.ds(ebase, E_PER_W)], cidx_v)

        def issue(c, b):
            pltpu.async_copy(
                out_hbm.at[ridx_v.at[pl.ds(c * E_CHK, E_CHK)]],
                rows_b[b], sem_g[b])
            pltpu.async_copy(
                out_hbm.at[cidx_v.at[pl.ds(c * E_CHK, E_CHK)]],
                cols_b[b], sem_g[b])

        def wait_and_accum(c, b, accs):
            pltpu.make_async_copy(
                out_hbm.at[ridx_v.at[pl.ds(c * E_CHK, E_CHK)]],
                rows_b[b], sem_g[b]).wait()
            pltpu.make_async_copy(
                out_hbm.at[cidx_v.at[pl.ds(c * E_CHK, E_CHK)]],
                cols_b[b], sem_g[b]).wait()

            def edge_body(e, a):
                new = []
                for j in range(VECS_PER_ROW):
                    r = rows_b[b][e, pl.ds(j * 16, 16)]
                    cc = cols_b[b][e, pl.ds(j * 16, 16)]
                    d = r - cc
                    new.append(a[j] + d * d)
                return tuple(new)

            return lax.fori_loop(0, E_CHK, edge_body, accs)

        issue(0, 0)

        def pair_body(g, accs):
            c0 = 2 * g
            issue(c0 + 1, 1)
            accs = wait_and_accum(c0, 0, accs)
            issue(c0 + 2, 0)
            accs = wait_and_accum(c0 + 1, 1, accs)
            return accs

        accs = lax.fori_loop(
            0, (N_CHUNKS - 1) // 2, pair_body,
            tuple(zero for _ in range(VECS_PER_ROW)))
        accs = wait_and_accum(N_CHUNKS - 1, 0, accs)

        total = accs[0]
        for j in range(1, VECS_PER_ROW):
            total = total + accs[j]
        smstage_v[...] = total
        pltpu.sync_copy(smstage_v, smooth_out.at[wid])

        for k in range(SUP_ITERS):
            pltpu.make_async_copy(
                tgt_hbm.at[pl.ds(0, SUP_CHK)],
                suptgt_v.at[pl.ds(k * SUP_CHK, SUP_CHK)], sem_s).wait()
            pltpu.make_async_copy(
                out_hbm.at[pl.ds(0, SUP_CHK)],
                suprows_v.at[pl.ds(k * SUP_CHK, SUP_CHK)], sem_s).wait()

        def sup_body(k, acc):
            chunk = wid + NW * k
            valid = chunk < N_SUP_CHUNKS
            tvec = suptgt_v[pl.ds(k * SUP_CHK, SUP_CHK)]
            contrib = jnp.zeros((16,), jnp.float32)
            for i in range(SUP_CHK):
                t = tvec[i]
                for j in range(VECS_PER_ROW):
                    blk = suprows_v[k * SUP_CHK + i, pl.ds(j * 16, 16)]
                    contrib = contrib + jnp.where(lanes + j * 16 == t, blk, 0.0)
            return acc + jnp.where(valid, contrib, 0.0)

        sup_acc = lax.fori_loop(0, SUP_ITERS, sup_body, zero)
        supstage_v[...] = sup_acc
        pltpu.sync_copy(supstage_v, sup_out.at[wid])

    return graph_loss_kernel


_graph_loss = _make_kernel()


@jax.jit
def kernel(output, target, edge_index):
    row = edge_index[0]
    col = edge_index[1]
    sup_p, smooth_p = _graph_loss(output, target, row, col)
    supervised = -jnp.sum(sup_p) / N_NODES
    smoothness = jnp.sum(smooth_p) / (N_EDGES * NUM_CLASSES)
    return supervised + MU * smoothness


# speedup vs baseline: 9.1816x; 1.0556x over previous
import functools

import jax
import jax.numpy as jnp
from jax import lax
from jax.experimental import pallas as pl
from jax.experimental.pallas import tpu as pltpu
from jax.experimental.pallas import tpu_sc as plsc

N_NODES = 10000
NUM_CLASSES = 128
N_EDGES = 320000
MU = 0.1

NC = 2
NS = 16
NW = NC * NS

E_PER_W = N_EDGES // NW
E_CHK = 80
N_CHUNKS = E_PER_W // E_CHK

SUP_CHK = 16
N_SUP_CHUNKS = N_NODES // SUP_CHK
SUP_ITERS = (N_SUP_CHUNKS + NW - 1) // NW
SUP_ROWS = SUP_ITERS * SUP_CHK

VECS_PER_ROW = NUM_CLASSES // 16


def _make_kernel():
    mesh = plsc.VectorSubcoreMesh(core_axis_name="c", subcore_axis_name="s")

    @functools.partial(
        pl.kernel,
        mesh=mesh,
        compiler_params=pltpu.CompilerParams(use_tc_tiling_on_sc=False),
        out_type=[
            jax.ShapeDtypeStruct((NW, 16), jnp.float32),
            jax.ShapeDtypeStruct((NW, 16), jnp.float32),
        ],
        scratch_types=[
            pltpu.VMEM((E_PER_W,), jnp.int32),
            pltpu.VMEM((E_PER_W,), jnp.int32),
            pltpu.VMEM((E_CHK, NUM_CLASSES // 2), jnp.int32),
            pltpu.VMEM((E_CHK, NUM_CLASSES // 2), jnp.int32),
            pltpu.VMEM((E_CHK, NUM_CLASSES // 2), jnp.int32),
            pltpu.VMEM((E_CHK, NUM_CLASSES // 2), jnp.int32),
            pltpu.VMEM((SUP_ROWS, NUM_CLASSES), jnp.float32),
            pltpu.VMEM((SUP_ROWS,), jnp.int32),
            pltpu.VMEM((16,), jnp.float32),
            pltpu.VMEM((16,), jnp.float32),
            pltpu.SemaphoreType.DMA,
            pltpu.SemaphoreType.DMA,
            pltpu.SemaphoreType.DMA,
        ],
    )
    def graph_loss_kernel(out_hbm, outbf_hbm, tgt_hbm, row_hbm, col_hbm,
                          sup_out, smooth_out,
                          ridx_v, cidx_v, rows0_v, rows1_v, cols0_v, cols1_v,
                          suprows_v, suptgt_v, supstage_v, smstage_v,
                          sem_s, sem_g0, sem_g1):
        wid = lax.axis_index("s") * NC + lax.axis_index("c")
        lanes = lax.iota(jnp.int32, 16)
        zero = jnp.zeros((16,), jnp.float32)
        rows_b = (rows0_v, rows1_v)
        cols_b = (cols0_v, cols1_v)
        sem_g = (sem_g0, sem_g1)

        for k in range(SUP_ITERS):
            chunk = wid + NW * k
            base = jnp.minimum(chunk, N_SUP_CHUNKS - 1) * SUP_CHK
            pltpu.async_copy(
                tgt_hbm.at[pl.ds(base, SUP_CHK)],
                suptgt_v.at[pl.ds(k * SUP_CHK, SUP_CHK)], sem_s)
            pltpu.async_copy(
                out_hbm.at[pl.ds(base, SUP_CHK)],
                suprows_v.at[pl.ds(k * SUP_CHK, SUP_CHK)], sem_s)

        ebase = wid * E_PER_W
        pltpu.sync_copy(row_hbm.at[pl.ds(ebase, E_PER_W)], ridx_v)
        pltpu.sync_copy(col_hbm.at[pl.ds(ebase, E_PER_W)], cidx_v)

        def issue(c, b):
            pltpu.async_copy(
                outbf_hbm.at[ridx_v.at[pl.ds(c * E_CHK, E_CHK)]],
                rows_b[b], sem_g[b])
            pltpu.async_copy(
                outbf_hbm.at[cidx_v.at[pl.ds(c * E_CHK, E_CHK)]],
                cols_b[b], sem_g[b])

        def wait_and_accum(c, b, accs):
            pltpu.make_async_copy(
                outbf_hbm.at[ridx_v.at[pl.ds(c * E_CHK, E_CHK)]],
                rows_b[b], sem_g[b]).wait()
            pltpu.make_async_copy(
                outbf_hbm.at[cidx_v.at[pl.ds(c * E_CHK, E_CHK)]],
                cols_b[b], sem_g[b]).wait()

            def edge_body(e, a):
                new = list(a)
                hi_mask = jnp.full((16,), -65536, jnp.int32)
                for j in range(VECS_PER_ROW // 2):
                    r = rows_b[b][e, pl.ds(j * 16, 16)]
                    cc = cols_b[b][e, pl.ds(j * 16, 16)]
                    r_lo = lax.bitcast_convert_type(r << 16, jnp.float32)
                    r_hi = lax.bitcast_convert_type(r & hi_mask, jnp.float32)
                    c_lo = lax.bitcast_convert_type(cc << 16, jnp.float32)
                    c_hi = lax.bitcast_convert_type(cc & hi_mask, jnp.float32)
                    dl = r_lo - c_lo
                    dh = r_hi - c_hi
                    new[2 * j] = new[2 * j] + dl * dl
                    new[2 * j + 1] = new[2 * j + 1] + dh * dh
                return tuple(new)

            return lax.fori_loop(0, E_CHK, edge_body, accs)

        issue(0, 0)

        def pair_body(g, accs):
            c0 = 2 * g
            issue(c0 + 1, 1)
            accs = wait_and_accum(c0, 0, accs)
            issue(c0 + 2, 0)
            accs = wait_and_accum(c0 + 1, 1, accs)
            return accs

        accs = lax.fori_loop(
            0, (N_CHUNKS - 1) // 2, pair_body,
            tuple(zero for _ in range(VECS_PER_ROW)))
        accs = wait_and_accum(N_CHUNKS - 1, 0, accs)

        total = accs[0]
        for j in range(1, VECS_PER_ROW):
            total = total + accs[j]
        smstage_v[...] = total
        pltpu.sync_copy(smstage_v, smooth_out.at[wid])

        for k in range(SUP_ITERS):
            pltpu.make_async_copy(
                tgt_hbm.at[pl.ds(0, SUP_CHK)],
                suptgt_v.at[pl.ds(k * SUP_CHK, SUP_CHK)], sem_s).wait()
            pltpu.make_async_copy(
                out_hbm.at[pl.ds(0, SUP_CHK)],
                suprows_v.at[pl.ds(k * SUP_CHK, SUP_CHK)], sem_s).wait()

        def sup_body(k, acc):
            chunk = wid + NW * k
            valid = chunk < N_SUP_CHUNKS
            tvec = suptgt_v[pl.ds(k * SUP_CHK, SUP_CHK)]
            contrib = jnp.zeros((16,), jnp.float32)
            for i in range(SUP_CHK):
                t = tvec[i]
                for j in range(VECS_PER_ROW):
                    blk = suprows_v[k * SUP_CHK + i, pl.ds(j * 16, 16)]
                    contrib = contrib + jnp.where(lanes + j * 16 == t, blk, 0.0)
            return acc + jnp.where(valid, contrib, 0.0)

        sup_acc = lax.fori_loop(0, SUP_ITERS, sup_body, zero)
        supstage_v[...] = sup_acc
        pltpu.sync_copy(supstage_v, sup_out.at[wid])

    return graph_loss_kernel


_graph_loss = _make_kernel()


@jax.jit
def kernel(output, target, edge_index):
    row = edge_index[0]
    col = edge_index[1]
    outpk = lax.bitcast_convert_type(
        output.astype(jnp.bfloat16).reshape(N_NODES, NUM_CLASSES // 2, 2),
        jnp.int32)
    sup_p, smooth_p = _graph_loss(output, outpk, target, row, col)
    supervised = -jnp.sum(sup_p) / N_NODES
    smoothness = jnp.sum(smooth_p) / (N_EDGES * NUM_CLASSES)
    return supervised + MU * smoothness
